# Initial kernel scaffold; baseline (speedup 1.0000x reference)
import jax
import jax.numpy as jnp
from jax.experimental import pallas as pl


def kernel(x, seg, word_table, pos_table, seg_table):
    raise NotImplementedError("write your pallas kernel here")


# speedup vs baseline: 1.5699x; 1.5699x over previous
import functools

import jax
import jax.numpy as jnp
from jax import lax
from jax.experimental import pallas as pl
from jax.experimental.pallas import tpu as pltpu
from jax.experimental.pallas import tpu_sc as plsc

B = 1024
L = 512
D = 768
NC = 2
NS = 16
NW = NC * NS
N_TOK = B * L
TOK_PER_W = N_TOK // NW
C = 64
N_CHUNK = TOK_PER_W // C
DSL = D // 16


def _sc_body(x_hbm, seg_hbm, word_hbm, combo_hbm, out_hbm,
             idx_v, seg_v, cidx_v, rows_v, add_v, sem_w, sem_c):
    wid = lax.axis_index("s") * NC + lax.axis_index("c")
    base = wid * TOK_PER_W

    def chunk(c, carry):
        t0 = base + c * C
        pltpu.sync_copy(x_hbm.at[pl.ds(t0, C)], idx_v)
        pltpu.sync_copy(seg_hbm.at[pl.ds(t0, C)], seg_v)
        word_cp = pltpu.async_copy(word_hbm.at[idx_v], rows_v, sem_w)
        p0 = lax.rem(c, L // C) * C
        for u in range(C // 16):
            s16 = seg_v[pl.ds(u * 16, 16)]
            l16 = lax.iota(jnp.int32, 16) + (p0 + u * 16)
            cidx_v[pl.ds(u * 16, 16)] = s16 * L + l16
        combo_cp = pltpu.async_copy(combo_hbm.at[cidx_v], add_v, sem_c)
        word_cp.wait()
        combo_cp.wait()

        def per_row(ci, _):
            for j in range(DSL):
                plsc.addupdate(rows_v.at[ci, pl.ds(j * 16, 16)],
                               add_v[ci, pl.ds(j * 16, 16)])
            return _

        lax.fori_loop(0, C, per_row, 0)
        pltpu.sync_copy(rows_v, out_hbm.at[pl.ds(t0, C)])
        return carry

    lax.fori_loop(0, N_CHUNK, chunk, 0)


@functools.partial(jax.jit, static_argnames=())
def _sc_embed(x_flat, seg_flat, word_table, combo):
    mesh = plsc.VectorSubcoreMesh(core_axis_name="c", subcore_axis_name="s",
                                  num_cores=NC, num_subcores=NS)
    f = pl.kernel(
        _sc_body,
        out_type=jax.ShapeDtypeStruct((N_TOK, D), jnp.float32),
        mesh=mesh,
        scratch_types=[
            pltpu.VMEM((C,), jnp.int32),
            pltpu.VMEM((C,), jnp.int32),
            pltpu.VMEM((C,), jnp.int32),
            pltpu.VMEM((C, D), jnp.float32),
            pltpu.VMEM((C, D), jnp.float32),
            pltpu.SemaphoreType.DMA,
            pltpu.SemaphoreType.DMA,
        ],
    )
    return f(x_flat, seg_flat, word_table, combo)


def _mask_body(x_ref, o_ref):
    o_ref[...] = x_ref[...] > 0


def _mask(x):
    return pl.pallas_call(
        _mask_body,
        out_shape=jax.ShapeDtypeStruct((B, L), jnp.bool_),
        grid=(8,),
        in_specs=[pl.BlockSpec((B // 8, L), lambda i: (i, 0))],
        out_specs=pl.BlockSpec((B // 8, L), lambda i: (i, 0)),
    )(x)


def kernel(x, seg, word_table, pos_table, seg_table):
    combo = (seg_table[:, None, :] + pos_table[None, :, :]).reshape(2 * L, D)
    out_flat = _sc_embed(x.reshape(N_TOK), seg.reshape(N_TOK), word_table, combo)
    return out_flat.reshape(B, L, D), _mask(x)
